# baseline (speedup 1.0000x reference)
import functools

import jax
import jax.numpy as jnp
from jax.experimental import pallas as pl

_STRIDES = (8, 16, 32, 64, 128)
_LEVEL_SIZES = (4096, 1024, 256, 64, 16)
_NUM_POINTS = sum(_LEVEL_SIZES)
_NUM_GT = 200
_NUM_CLASSES = 80
_TOPK = 9
_GBLK = 128


def _center_prior_kernel(lvl0_ref, lvl1_ref, lvl2_ref, lvl3_ref, lvl4_ref,
                         gtb_t_ref, msig_t_ref, labels_ref, inside_ref,
                         out_w_ref, out_m_ref):
    n = _NUM_POINTS
    g = _GBLK
    j = pl.program_id(0)

    gtb = gtb_t_ref[...]
    cx = (gtb[0:1, :] + gtb[2:3, :]) * 0.5
    cy = (gtb[1:2, :] + gtb[3:4, :]) * 0.5

    lab = labels_ref[...]
    cls_iota = jax.lax.broadcasted_iota(jnp.int32, (_NUM_CLASSES, g), 0)
    oh = (cls_iota == lab).astype(jnp.float32)
    msig = jax.lax.dot_general(
        msig_t_ref[...], oh, (((1,), (0,)), ((), ())),
        precision=jax.lax.Precision.HIGHEST,
        preferred_element_type=jnp.float32)
    mx = msig[0:1, :]
    my = msig[1:2, :]
    nax = -1.0 / (2.0 * msig[2:3, :] * msig[2:3, :])
    nay = -1.0 / (2.0 * msig[3:4, :] * msig[3:4, :])

    w_chunks = []
    for lvl_ref, s in zip((lvl0_ref, lvl1_ref, lvl2_ref, lvl3_ref, lvl4_ref),
                          _STRIDES):
        inv = 1.0 / s
        pts = lvl_ref[...]
        ax = pts[:, 0:1] * inv
        ay = pts[:, 1:2] * inv
        bx = cx * inv + mx
        by = cy * inv + my
        dx = ax - bx
        dy = ay - by
        w_chunks.append(jnp.exp(dx * dx * nax + dy * dy * nay))
    w = jnp.concatenate(w_chunks, axis=0)

    inside = jnp.where(inside_ref[...], 1.0, 0.0)
    cnt = jnp.sum(inside, axis=0, keepdims=True)
    col_valid = jax.lax.broadcasted_iota(jnp.int32, (1, g), 1) + j * g < _NUM_GT
    no_pts = jnp.logical_and(cnt == 0.0, col_valid)
    need_topk = jnp.any(no_pts)

    @pl.when(need_topk)
    def _force_topk_path():
        row_iota = jax.lax.broadcasted_iota(jnp.int32, (n, g), 0)
        cur = w
        for _ in range(_TOPK):
            idx = jnp.argmax(cur, axis=0, keepdims=True)
            cur = jnp.where(row_iota == idx, -1.0, cur)
        taken = jnp.where(cur < 0.0, 1.0, 0.0)
        inside_mask = jnp.where(no_pts, taken, inside)
        out_m_ref[...] = inside_mask > 0.5
        out_w_ref[...] = w * inside_mask

    @pl.when(jnp.logical_not(need_topk))
    def _plain_path():
        out_m_ref[...] = inside > 0.5
        out_w_ref[...] = w * inside


@functools.partial(jax.jit, static_argnames=())
def kernel(anchor_points_lvl0, anchor_points_lvl1, anchor_points_lvl2,
           anchor_points_lvl3, anchor_points_lvl4, gt_bboxes, mean, sigma,
           labels, inside_gt_bbox_mask):
    gtb_t = gt_bboxes.T
    msig_t = jnp.concatenate([mean.T, sigma.T], axis=0)
    labels2 = labels.reshape(1, _NUM_GT).astype(jnp.int32)

    nblk = (_NUM_GT + _GBLK - 1) // _GBLK
    lvl_specs = [
        pl.BlockSpec((sz, 2), lambda j: (0, 0)) for sz in _LEVEL_SIZES
    ]
    out_w, out_m = pl.pallas_call(
        _center_prior_kernel,
        grid=(nblk,),
        in_specs=lvl_specs + [
            pl.BlockSpec((4, _GBLK), lambda j: (0, j)),
            pl.BlockSpec((4, _NUM_CLASSES), lambda j: (0, 0)),
            pl.BlockSpec((1, _GBLK), lambda j: (0, j)),
            pl.BlockSpec((_NUM_POINTS, _GBLK), lambda j: (0, j)),
        ],
        out_specs=(
            pl.BlockSpec((_NUM_POINTS, _GBLK), lambda j: (0, j)),
            pl.BlockSpec((_NUM_POINTS, _GBLK), lambda j: (0, j)),
        ),
        out_shape=(
            jax.ShapeDtypeStruct((_NUM_POINTS, _NUM_GT), jnp.float32),
            jax.ShapeDtypeStruct((_NUM_POINTS, _NUM_GT), jnp.bool_),
        ),
    )(anchor_points_lvl0, anchor_points_lvl1, anchor_points_lvl2,
      anchor_points_lvl3, anchor_points_lvl4, gtb_t, msig_t, labels2,
      inside_gt_bbox_mask)
    return out_w, out_m


# speedup vs baseline: 1.1390x; 1.1390x over previous
import functools

import jax
import jax.numpy as jnp
from jax.experimental import pallas as pl

_STRIDES = (8, 16, 32, 64, 128)
_LEVEL_SIZES = (4096, 1024, 256, 64, 16)
_NUM_POINTS = sum(_LEVEL_SIZES)
_NUM_GT = 200
_NUM_CLASSES = 80
_TOPK = 9
_GBLK = 128


def _center_prior_kernel(pts_ref, gtb_t_ref, msig_t_ref, labels_ref,
                         inside_ref, out_w_ref, out_m_ref):
    n = _NUM_POINTS
    g = _GBLK
    j = pl.program_id(0)

    gtb = gtb_t_ref[...]
    cx = (gtb[0:1, :] + gtb[2:3, :]) * 0.5
    cy = (gtb[1:2, :] + gtb[3:4, :]) * 0.5

    lab = labels_ref[...]
    cls_iota = jax.lax.broadcasted_iota(jnp.int32, (_NUM_CLASSES, g), 0)
    oh = (cls_iota == lab).astype(jnp.float32)
    msig = jax.lax.dot_general(
        msig_t_ref[...], oh, (((1,), (0,)), ((), ())),
        precision=jax.lax.Precision.HIGHEST,
        preferred_element_type=jnp.float32)
    mx = msig[0:1, :]
    my = msig[1:2, :]
    nax = -1.0 / (2.0 * msig[2:3, :] * msig[2:3, :])
    nay = -1.0 / (2.0 * msig[3:4, :] * msig[3:4, :])

    w_chunks = []
    base = 0
    for sz, s in zip(_LEVEL_SIZES, _STRIDES):
        inv = 1.0 / s
        pts = pts_ref[base:base + sz, :]
        base += sz
        ax = pts[:, 0:1] * inv
        ay = pts[:, 1:2] * inv
        bx = cx * inv + mx
        by = cy * inv + my
        dx = ax - bx
        dy = ay - by
        w_chunks.append(jnp.exp(dx * dx * nax + dy * dy * nay))
    w = jnp.concatenate(w_chunks, axis=0)

    inside = jnp.where(inside_ref[...], 1.0, 0.0)
    cnt = jnp.sum(inside, axis=0, keepdims=True)
    col_valid = jax.lax.broadcasted_iota(jnp.int32, (1, g), 1) + j * g < _NUM_GT
    no_pts = jnp.logical_and(cnt == 0.0, col_valid)
    need_topk = jnp.any(no_pts)

    @pl.when(need_topk)
    def _force_topk_path():
        row_iota = jax.lax.broadcasted_iota(jnp.int32, (n, g), 0)
        cur = w
        for _ in range(_TOPK):
            idx = jnp.argmax(cur, axis=0, keepdims=True)
            cur = jnp.where(row_iota == idx, -1.0, cur)
        taken = jnp.where(cur < 0.0, 1.0, 0.0)
        inside_mask = jnp.where(no_pts, taken, inside)
        out_m_ref[...] = inside_mask > 0.5
        out_w_ref[...] = w * inside_mask

    @pl.when(jnp.logical_not(need_topk))
    def _plain_path():
        out_m_ref[...] = inside > 0.5
        out_w_ref[...] = w * inside


@functools.partial(jax.jit, static_argnames=())
def kernel(anchor_points_lvl0, anchor_points_lvl1, anchor_points_lvl2,
           anchor_points_lvl3, anchor_points_lvl4, gt_bboxes, mean, sigma,
           labels, inside_gt_bbox_mask):
    pts_all = jnp.concatenate([anchor_points_lvl0, anchor_points_lvl1,
                               anchor_points_lvl2, anchor_points_lvl3,
                               anchor_points_lvl4], axis=0)
    gtb_t = gt_bboxes.T
    msig_t = jnp.concatenate([mean.T, sigma.T], axis=0)
    labels2 = labels.reshape(1, _NUM_GT).astype(jnp.int32)

    nblk = (_NUM_GT + _GBLK - 1) // _GBLK
    out_w, out_m = pl.pallas_call(
        _center_prior_kernel,
        grid=(nblk,),
        in_specs=[
            pl.BlockSpec((_NUM_POINTS, 2), lambda j: (0, 0)),
            pl.BlockSpec((4, _GBLK), lambda j: (0, j)),
            pl.BlockSpec((4, _NUM_CLASSES), lambda j: (0, 0)),
            pl.BlockSpec((1, _GBLK), lambda j: (0, j)),
            pl.BlockSpec((_NUM_POINTS, _GBLK), lambda j: (0, j)),
        ],
        out_specs=(
            pl.BlockSpec((_NUM_POINTS, _GBLK), lambda j: (0, j)),
            pl.BlockSpec((_NUM_POINTS, _GBLK), lambda j: (0, j)),
        ),
        out_shape=(
            jax.ShapeDtypeStruct((_NUM_POINTS, _NUM_GT), jnp.float32),
            jax.ShapeDtypeStruct((_NUM_POINTS, _NUM_GT), jnp.bool_),
        ),
    )(pts_all, gtb_t, msig_t, labels2, inside_gt_bbox_mask)
    return out_w, out_m
